# baseline (speedup 1.0000x reference)
import functools

import jax
import jax.numpy as jnp
from jax import lax
from jax.experimental import pallas as pl
from jax.experimental.pallas import tpu as pltpu
from jax.experimental.pallas import tpu_sc as plsc

N = 10000
E = 320000
F_IN = 128
HID = 128
NCLS = 32
DE = 16
G = 64

N_PAD = 10240
E_PAD = 327680
C = 64
NTILES = 32
EPT = E_PAD // NTILES
NCH = EPT // C
RPT = N_PAD // 16
C2 = 128
EPT2 = E_PAD // 16
NCH2 = EPT2 // C2

_f32 = jnp.float32
_i32 = jnp.int32


def _sc_mesh():
    return plsc.VectorSubcoreMesh(core_axis_name="c", subcore_axis_name="s",
                                  num_cores=2, num_subcores=16)


def _sc_edge_attr_mean(edge_attr, didx, interpret=False):
    kfn = functools.partial(
        pl.kernel,
        out_type=(
            jax.ShapeDtypeStruct((2, N_PAD, DE), _f32),
            jax.ShapeDtypeStruct((NTILES, N_PAD), _f32),
        ),
        mesh=_sc_mesh(),
        compiler_params=pltpu.CompilerParams(needs_layout_passes=False, use_tc_tiling_on_sc=False),
        scratch_types=[
            pltpu.VMEM((NCH, C), _i32),
            pltpu.VMEM((C, DE), _f32),
            pltpu.VMEM((N_PAD,), _f32),
            pltpu.VMEM((16,), _f32),
            pltpu.MemorySpace.VMEM_SHARED((N_PAD, DE), _f32),
        ],
        interpret=interpret,
    )

    @kfn
    def k(ea_hbm, didx_hbm, lsum_out, lcnt_out,
          didx_t, ebuf, cnt_t, ones_v, lsum_sh):
        cid = lax.axis_index("c")
        sid = lax.axis_index("s")
        w = cid * 16 + sid

        def _z_ebuf(i, _):
            ebuf[i, :] = jnp.zeros((DE,), _f32)
            return 0
        lax.fori_loop(0, C, _z_ebuf, 0)

        def _z_cnt(i, _):
            cnt_t[pl.ds(i * 16, 16)] = jnp.zeros((16,), _f32)
            return 0
        lax.fori_loop(0, N_PAD // 16, _z_cnt, 0)
        ones_v[...] = jnp.ones((16,), _f32)

        def _z_sh(i, _):
            pltpu.sync_copy(ebuf, lsum_sh.at[pl.ds(sid * RPT + i * C, C)])
            return 0
        lax.fori_loop(0, RPT // C, _z_sh, 0)
        plsc.subcore_barrier()

        pltpu.sync_copy(didx_hbm.at[w], didx_t)

        def _chunk(kk, _):
            base = w * EPT + kk * C
            pltpu.sync_copy(ea_hbm.at[pl.ds(base, C)], ebuf)
            pltpu.sync_copy(ebuf, lsum_sh.at[didx_t.at[kk]], add=True)
            for g in range(C // 16):
                d16 = didx_t[kk, pl.ds(g * 16, 16)]
                plsc.addupdate_scatter(cnt_t, [d16], ones_v[...])
            return 0
        lax.fori_loop(0, NCH, _chunk, 0)

        plsc.subcore_barrier()
        pltpu.sync_copy(lsum_sh.at[pl.ds(sid * RPT, RPT)],
                        lsum_out.at[cid, pl.ds(sid * RPT, RPT)])
        pltpu.sync_copy(cnt_t, lcnt_out.at[w])

    return k(edge_attr, didx)


def _sc_edge_aggregate(hcat, asrc, adst, eidx, F, interpret=False):
    F2 = F // 2
    kfn = functools.partial(
        pl.kernel,
        out_type=(
            jax.ShapeDtypeStruct((2, N_PAD, F2), _f32),
            jax.ShapeDtypeStruct((NTILES, N_PAD), _f32),
        ),
        mesh=_sc_mesh(),
        compiler_params=pltpu.CompilerParams(needs_layout_passes=False, use_tc_tiling_on_sc=False),
        scratch_types=[
            pltpu.VMEM((N_PAD,), _f32),
            pltpu.VMEM((N_PAD,), _f32),
            pltpu.VMEM((N_PAD,), _f32),
            pltpu.VMEM((3, 3, C2), _i32),
            pltpu.VMEM((2, C2), _i32),
            pltpu.VMEM((2, C2, F2), _f32),
            pltpu.VMEM((C2,), _f32),
            pltpu.MemorySpace.VMEM_SHARED((N_PAD, F2), _f32),
            pltpu.SemaphoreType.DMA((3,)),
            pltpu.SemaphoreType.DMA((2,)),
            pltpu.SemaphoreType.DMA((2,)),
        ],
        interpret=interpret,
    )

    @kfn
    def k(h_hbm, asrc_hbm, adst_hbm, eidx_hbm,
          num_out, den_out,
          asrc_t, adst_t, den_t, idxb, gidx, rows, wbuf,
          num_sh, isem, gsem, ssem):
        cid = lax.axis_index("c")
        sid = lax.axis_index("s")
        w = cid * 16 + sid
        bias = cid * N_PAD

        def _z_rows(i, _):
            for j in range(F2 // 16):
                rows[0, i, pl.ds(j * 16, 16)] = jnp.zeros((16,), _f32)
            return 0
        lax.fori_loop(0, C2, _z_rows, 0)

        def _z_den(i, _):
            den_t[pl.ds(i * 16, 16)] = jnp.zeros((16,), _f32)
            return 0
        lax.fori_loop(0, N_PAD // 16, _z_den, 0)

        def _z_sh(i, _):
            pltpu.sync_copy(rows.at[0],
                            num_sh.at[pl.ds(sid * RPT + i * C2, C2)])
            return 0
        lax.fori_loop(0, RPT // C2, _z_sh, 0)
        plsc.subcore_barrier()

        pltpu.sync_copy(asrc_hbm, asrc_t)
        pltpu.sync_copy(adst_hbm, adst_t)

        def _load_idx(kk, r):
            pltpu.async_copy(eidx_hbm.at[sid, kk], idxb.at[r], isem.at[r])

        def _bias_idx(r, b):
            for j in range(C2 // 16):
                gidx[b, pl.ds(j * 16, 16)] = (
                    idxb[r, 0, pl.ds(j * 16, 16)] + bias)

        _load_idx(0, 0)
        _load_idx(1, 1)
        pltpu.make_async_copy(eidx_hbm.at[sid, 0], idxb.at[0], isem.at[0]).wait()
        _bias_idx(0, 0)
        pltpu.async_copy(h_hbm.at[gidx.at[0]], rows.at[0], gsem.at[0])

        def _chunk(kk, _):
            p = lax.rem(kk, 2)
            q = 1 - p
            r = lax.rem(kk, 3)

            pltpu.make_async_copy(
                h_hbm.at[gidx.at[p]], rows.at[p], gsem.at[p]).wait()

            for g in range(C2 // 16):
                s16 = idxb[r, 0, pl.ds(g * 16, 16)]
                d16 = idxb[r, 1, pl.ds(g * 16, 16)]
                a = (plsc.load_gather(asrc_t, [s16])
                     + plsc.load_gather(adst_t, [d16])
                     + plsc.bitcast(idxb[r, 2, pl.ds(g * 16, 16)], _f32))
                a = jnp.maximum(a, 0.2 * a)
                wv = jnp.exp(a)
                wbuf[pl.ds(g * 16, 16)] = wv
                plsc.addupdate_scatter(den_t, [d16], wv)

            UN = 16
            def _scale(e8, _):
                base = e8 * UN
                ws = [plsc.load_gather(wbuf, [jnp.full((16,), base + u, _i32)])
                      for u in range(UN)]
                for u in range(UN):
                    for j in range(F2 // 16):
                        rows[p, base + u, pl.ds(j * 16, 16)] = (
                            rows[p, base + u, pl.ds(j * 16, 16)] * ws[u])
                return 0
            lax.fori_loop(0, C2 // UN, _scale, 0)

            pltpu.async_copy(rows.at[p], num_sh.at[idxb.at[r, 1]], ssem.at[p],
                             add=True)

            @pl.when(kk + 1 < NCH2)
            def _():
                r1 = lax.rem(kk + 1, 3)
                pltpu.make_async_copy(
                    eidx_hbm.at[sid, kk], idxb.at[r1], isem.at[r1]).wait()

                @pl.when(kk >= 1)
                def _():
                    pltpu.make_async_copy(
                        rows.at[q], num_sh.at[idxb.at[r1, 1]],
                        ssem.at[q]).wait()
                _bias_idx(r1, q)
                pltpu.async_copy(h_hbm.at[gidx.at[q]], rows.at[q],
                                 gsem.at[q])

            @pl.when(kk + 2 < NCH2)
            def _():
                _load_idx(kk + 2, lax.rem(kk + 2, 3))
            return 0
        lax.fori_loop(0, NCH2, _chunk, 0)

        for b in (0, 1):
            pltpu.make_async_copy(
                rows.at[b], num_sh.at[idxb.at[0, 1]], ssem.at[b]).wait()

        plsc.subcore_barrier()
        pltpu.sync_copy(num_sh.at[pl.ds(sid * RPT, RPT)],
                        num_out.at[cid, pl.ds(sid * RPT, RPT)])
        pltpu.sync_copy(den_t, den_out.at[w])

    return k(hcat, asrc, adst, eidx)


def _tc_matmul_scores(x, W, UV, F, interpret=False):
    BN = 2048
    grid = (N_PAD // BN,)

    def body(x_ref, w_ref, uv_ref, h2_ref, sc_ref):
        xb = x_ref[...]
        hb = jnp.dot(xb, w_ref[...], preferred_element_type=_f32)
        F2 = hb.shape[1] // 2
        h2_ref[0] = hb[:, :F2]
        h2_ref[1] = hb[:, F2:]
        sc_ref[...] = lax.dot_general(
            uv_ref[...], xb, (((1,), (1,)), ((), ())),
            preferred_element_type=_f32)

    return pl.pallas_call(
        body,
        grid=grid,
        in_specs=[
            pl.BlockSpec((BN, x.shape[1]), lambda i: (i, 0)),
            pl.BlockSpec(W.shape, lambda i: (0, 0)),
            pl.BlockSpec(UV.shape, lambda i: (0, 0)),
        ],
        out_specs=[
            pl.BlockSpec((2, BN, F // 2), lambda i: (0, i, 0)),
            pl.BlockSpec((8, BN), lambda i: (0, i)),
        ],
        out_shape=[
            jax.ShapeDtypeStruct((2, N_PAD, F // 2), _f32),
            jax.ShapeDtypeStruct((8, N_PAD), _f32),
        ],
        interpret=interpret,
    )(x, W, UV)


def _tc_loop_scores(lsum, lcnt, Vs, interpret=False):
    BN = 2048
    grid = (N_PAD // BN,)

    def body(ls_ref, lc_ref, vs_ref, out_ref):
        ls = ls_ref[0] + ls_ref[1]
        lc = jnp.sum(lc_ref[...], axis=0)
        la = ls / jnp.maximum(lc, 1.0)[:, None]
        out_ref[...] = lax.dot_general(
            vs_ref[...], la, (((1,), (1,)), ((), ())),
            preferred_element_type=_f32)

    return pl.pallas_call(
        body,
        grid=grid,
        in_specs=[
            pl.BlockSpec((2, BN, DE), lambda i: (0, i, 0)),
            pl.BlockSpec((NTILES, BN), lambda i: (0, i)),
            pl.BlockSpec(Vs.shape, lambda i: (0, 0)),
        ],
        out_specs=pl.BlockSpec((8, BN), lambda i: (0, i)),
        out_shape=jax.ShapeDtypeStruct((8, N_PAD), _f32),
        interpret=interpret,
    )(lsum, lcnt, Vs)


def _tc_edge_scores(edge_attr, V3, interpret=False):
    BE = 16384
    grid = (E_PAD // BE,)

    def body(ea_ref, v_ref, out_ref):
        out_ref[...] = lax.dot_general(
            v_ref[...], ea_ref[...], (((1,), (1,)), ((), ())),
            preferred_element_type=_f32)

    return pl.pallas_call(
        body,
        grid=grid,
        in_specs=[
            pl.BlockSpec((BE, DE), lambda i: (i, 0)),
            pl.BlockSpec(V3.shape, lambda i: (0, 0)),
        ],
        out_specs=pl.BlockSpec((8, BE), lambda i: (0, i)),
        out_shape=jax.ShapeDtypeStruct((8, E_PAD), _f32),
        interpret=interpret,
    )(edge_attr, V3)


def _tc_epilogue(num, den, h, scores, aloopT, b, lidx, F, interpret=False):
    BN = 2048
    grid = (N_PAD // BN,)

    def body(num_ref, den_ref, h_ref, sc_ref, al_ref, b_ref, out_ref):
        a = sc_ref[0] + sc_ref[1] + al_ref[lidx]
        a = jnp.maximum(a, 0.2 * a)
        wl = jnp.exp(a)
        hb = jnp.concatenate([h_ref[0], h_ref[1]], axis=1)
        nm = (jnp.concatenate([num_ref[0], num_ref[1]], axis=1)
              + wl[:, None] * hb)
        dn = 0.5 * jnp.sum(den_ref[...], axis=0) + wl
        out = nm / dn[:, None] + b_ref[...]
        out_ref[...] = jnp.maximum(out, 0.0)

    return pl.pallas_call(
        body,
        grid=grid,
        in_specs=[
            pl.BlockSpec((2, BN, F // 2), lambda i: (0, i, 0)),
            pl.BlockSpec((NTILES, BN), lambda i: (0, i)),
            pl.BlockSpec((2, BN, F // 2), lambda i: (0, i, 0)),
            pl.BlockSpec((8, BN), lambda i: (0, i)),
            pl.BlockSpec((8, BN), lambda i: (0, i)),
            pl.BlockSpec((1, F), lambda i: (0, 0)),
        ],
        out_specs=pl.BlockSpec((BN, F), lambda i: (i, 0)),
        out_shape=jax.ShapeDtypeStruct((N_PAD, F), _f32),
        interpret=interpret,
    )(num, den, h, scores, aloopT, b)


def _tc_pool(h, batch3, interpret=False):
    BN = 1280
    grid_n = N_PAD // BN

    def body(h_ref, b_ref, out_ref, sacc, cacc):
        i = pl.program_id(0)
        bb = b_ref[0, 0, :]
        oh = (bb[:, None] == lax.broadcasted_iota(_i32, (BN, G), 1)).astype(_f32)
        s = lax.dot_general(oh, h_ref[...], (((0,), (0,)), ((), ())),
                            preferred_element_type=_f32)
        ones = jnp.ones((BN, NCLS), _f32)
        c = lax.dot_general(oh, ones, (((0,), (0,)), ((), ())),
                            preferred_element_type=_f32)

        @pl.when(i == 0)
        def _():
            sacc[...] = s
            cacc[...] = c

        @pl.when(i > 0)
        def _():
            sacc[...] = sacc[...] + s
            cacc[...] = cacc[...] + c

        @pl.when(i == grid_n - 1)
        def _():
            pooled = sacc[...] / jnp.maximum(cacc[...], 1.0)
            out_ref[...] = 1.0 / (1.0 + jnp.exp(-pooled))

    return pl.pallas_call(
        body,
        grid=(grid_n,),
        in_specs=[
            pl.BlockSpec((BN, NCLS), lambda i: (i, 0)),
            pl.BlockSpec((1, 1, BN), lambda i: (i, 0, 0)),
        ],
        out_specs=pl.BlockSpec((G, NCLS), lambda i: (0, 0)),
        out_shape=jax.ShapeDtypeStruct((G, NCLS), _f32),
        scratch_shapes=[
            pltpu.VMEM((G, NCLS), _f32),
            pltpu.VMEM((G, NCLS), _f32),
        ],
        interpret=interpret,
    )(h, batch3)


def _run(x, edge_index, edge_attr, batch, params, interpret=False):
    pad_i = jnp.full((E_PAD - E,), N_PAD - 1, _i32)
    src = jnp.concatenate([edge_index[0], pad_i])
    dst_p = jnp.concatenate([edge_index[1], pad_i])
    didx = dst_p.reshape(NTILES, NCH, C)
    ea_p = jnp.zeros((E_PAD, DE), _f32).at[:E].set(edge_attr)

    uvs = []
    V3 = jnp.zeros((8, DE), _f32)
    for l, (Wl, asl, adl, Wel, ael, bl) in enumerate(params):
        UV = (jnp.zeros((8, Wl.shape[0]), _f32)
              .at[0].set(Wl @ asl).at[1].set(Wl @ adl))
        uvs.append(UV)
        V3 = V3.at[l].set(Wel @ ael)

    xp = jnp.zeros((N_PAD, F_IN), _f32).at[:N].set(x)
    batch_p = jnp.full((N_PAD,), G, _i32).at[:N].set(batch)
    batch3 = batch_p.reshape(N_PAD // 1280, 1, 1280)

    lsum, lcnt = _sc_edge_attr_mean(ea_p, didx, interpret=interpret)
    aloopT = _tc_loop_scores(lsum, lcnt, V3, interpret=interpret)
    aeT = _tc_edge_scores(ea_p, V3, interpret=interpret)

    h_in = xp
    for l, (Wl, asl, adl, Wel, ael, bl) in enumerate(params):
        F = Wl.shape[1]
        h2, scores = _tc_matmul_scores(h_in, Wl, uvs[l], F, interpret=interpret)
        hcat = h2.reshape(2 * N_PAD, F // 2)
        ae_bits = lax.bitcast_convert_type(aeT[l], _i32)
        eidx = (jnp.stack([src, dst_p, ae_bits])
                .reshape(3, 16, NCH2, C2).transpose(1, 2, 0, 3))
        num, den = _sc_edge_aggregate(hcat, scores[0], scores[1], eidx, F,
                                      interpret=interpret)
        h_in = _tc_epilogue(num, den, h2, scores, aloopT, bl.reshape(1, F),
                            l, F, interpret=interpret)

    return _tc_pool(h_in, batch3, interpret=interpret)


def kernel(x, edge_index, edge_attr, batch, W1, as1, ad1, We1, ae1, b1,
           W2, as2, ad2, We2, ae2, b2, W3, as3, ad3, We3, ae3, b3):
    params = [
        (W1, as1, ad1, We1, ae1, b1),
        (W2, as2, ad2, We2, ae2, b2),
        (W3, as3, ad3, We3, ae3, b3),
    ]
    return _run(x, edge_index, edge_attr, batch, params)


# speedup vs baseline: 1.2848x; 1.2848x over previous
import functools

import jax
import jax.numpy as jnp
from jax import lax
from jax.experimental import pallas as pl
from jax.experimental.pallas import tpu as pltpu
from jax.experimental.pallas import tpu_sc as plsc

N = 10000
E = 320000
F_IN = 128
HID = 128
NCLS = 32
DE = 16
G = 64

N_PAD = 10240
E_PAD = 327680
C = 64
NTILES = 32
EPT = E_PAD // NTILES
NCH = EPT // C
RPT = N_PAD // 16
C2 = 128
EPT2 = E_PAD // 16
NCH2 = EPT2 // C2

_f32 = jnp.float32
_i32 = jnp.int32


def _sc_mesh():
    return plsc.VectorSubcoreMesh(core_axis_name="c", subcore_axis_name="s",
                                  num_cores=2, num_subcores=16)


def _sc_edge_attr_mean(edge_attr, didx, interpret=False):
    kfn = functools.partial(
        pl.kernel,
        out_type=(
            jax.ShapeDtypeStruct((2, N_PAD, DE), _f32),
            jax.ShapeDtypeStruct((NTILES, N_PAD), _f32),
        ),
        mesh=_sc_mesh(),
        compiler_params=pltpu.CompilerParams(needs_layout_passes=False, use_tc_tiling_on_sc=False),
        scratch_types=[
            pltpu.VMEM((NCH, C), _i32),
            pltpu.VMEM((C, DE), _f32),
            pltpu.VMEM((N_PAD,), _f32),
            pltpu.VMEM((16,), _f32),
            pltpu.MemorySpace.VMEM_SHARED((N_PAD, DE), _f32),
        ],
        interpret=interpret,
    )

    @kfn
    def k(ea_hbm, didx_hbm, lsum_out, lcnt_out,
          didx_t, ebuf, cnt_t, ones_v, lsum_sh):
        cid = lax.axis_index("c")
        sid = lax.axis_index("s")
        w = cid * 16 + sid

        def _z_ebuf(i, _):
            ebuf[i, :] = jnp.zeros((DE,), _f32)
            return 0
        lax.fori_loop(0, C, _z_ebuf, 0)

        def _z_cnt(i, _):
            cnt_t[pl.ds(i * 16, 16)] = jnp.zeros((16,), _f32)
            return 0
        lax.fori_loop(0, N_PAD // 16, _z_cnt, 0)
        ones_v[...] = jnp.ones((16,), _f32)

        def _z_sh(i, _):
            pltpu.sync_copy(ebuf, lsum_sh.at[pl.ds(sid * RPT + i * C, C)])
            return 0
        lax.fori_loop(0, RPT // C, _z_sh, 0)
        plsc.subcore_barrier()

        pltpu.sync_copy(didx_hbm.at[w], didx_t)

        def _chunk(kk, _):
            base = w * EPT + kk * C
            pltpu.sync_copy(ea_hbm.at[pl.ds(base, C)], ebuf)
            pltpu.sync_copy(ebuf, lsum_sh.at[didx_t.at[kk]], add=True)
            for g in range(C // 16):
                d16 = didx_t[kk, pl.ds(g * 16, 16)]
                plsc.addupdate_scatter(cnt_t, [d16], ones_v[...])
            return 0
        lax.fori_loop(0, NCH, _chunk, 0)

        plsc.subcore_barrier()
        pltpu.sync_copy(lsum_sh.at[pl.ds(sid * RPT, RPT)],
                        lsum_out.at[cid, pl.ds(sid * RPT, RPT)])
        pltpu.sync_copy(cnt_t, lcnt_out.at[w])

    return k(edge_attr, didx)


def _sc_edge_aggregate(hcat, asrc, adst, eidx, F, interpret=False):
    F2 = F // 2
    kfn = functools.partial(
        pl.kernel,
        out_type=(
            jax.ShapeDtypeStruct((2, N_PAD, F2), _f32),
            jax.ShapeDtypeStruct((NTILES, N_PAD), _f32),
        ),
        mesh=_sc_mesh(),
        compiler_params=pltpu.CompilerParams(needs_layout_passes=False, use_tc_tiling_on_sc=False),
        scratch_types=[
            pltpu.VMEM((N_PAD,), _f32),
            pltpu.VMEM((N_PAD,), _f32),
            pltpu.VMEM((N_PAD,), _f32),
            pltpu.VMEM((3, 3, C2), _i32),
            pltpu.VMEM((2, C2), _i32),
            pltpu.VMEM((2, C2, F2), _f32),
            pltpu.VMEM((C2,), _f32),
            pltpu.MemorySpace.VMEM_SHARED((N_PAD, F2), _f32),
            pltpu.SemaphoreType.DMA((3,)),
            pltpu.SemaphoreType.DMA((2,)),
            pltpu.SemaphoreType.DMA((2,)),
        ],
        interpret=interpret,
    )

    @kfn
    def k(h_hbm, asrc_hbm, adst_hbm, eidx_hbm,
          num_out, den_out,
          asrc_t, adst_t, den_t, idxb, gidx, rows, wbuf,
          num_sh, isem, gsem, ssem):
        cid = lax.axis_index("c")
        sid = lax.axis_index("s")
        w = cid * 16 + sid
        bias = cid * N_PAD

        def _z_rows(i, _):
            for j in range(F2 // 16):
                rows[0, i, pl.ds(j * 16, 16)] = jnp.zeros((16,), _f32)
            return 0
        lax.fori_loop(0, C2, _z_rows, 0)

        def _z_den(i, _):
            den_t[pl.ds(i * 16, 16)] = jnp.zeros((16,), _f32)
            return 0
        lax.fori_loop(0, N_PAD // 16, _z_den, 0)

        def _z_sh(i, _):
            pltpu.sync_copy(rows.at[0],
                            num_sh.at[pl.ds(sid * RPT + i * C2, C2)])
            return 0
        lax.fori_loop(0, RPT // C2, _z_sh, 0)
        plsc.subcore_barrier()

        pltpu.sync_copy(asrc_hbm, asrc_t)
        pltpu.sync_copy(adst_hbm, adst_t)

        def _load_idx(kk, r):
            pltpu.async_copy(eidx_hbm.at[sid, kk], idxb.at[r], isem.at[r])

        def _bias_idx(r, b):
            for j in range(C2 // 16):
                gidx[b, pl.ds(j * 16, 16)] = (
                    idxb[r, 0, pl.ds(j * 16, 16)] + bias)

        _load_idx(0, 0)
        _load_idx(1, 1)
        pltpu.make_async_copy(eidx_hbm.at[sid, 0], idxb.at[0], isem.at[0]).wait()
        _bias_idx(0, 0)
        pltpu.async_copy(h_hbm.at[gidx.at[0]], rows.at[0], gsem.at[0])

        def _chunk(kk, _):
            p = lax.rem(kk, 2)
            q = 1 - p
            r = lax.rem(kk, 3)

            @pl.when(kk + 1 < NCH2)
            def _():
                r1 = lax.rem(kk + 1, 3)
                pltpu.make_async_copy(
                    eidx_hbm.at[sid, kk], idxb.at[r1], isem.at[r1]).wait()

                @pl.when(kk >= 1)
                def _():
                    pltpu.make_async_copy(
                        rows.at[q], num_sh.at[idxb.at[r1, 1]],
                        ssem.at[q]).wait()
                _bias_idx(r1, q)
                pltpu.async_copy(h_hbm.at[gidx.at[q]], rows.at[q],
                                 gsem.at[q])

            @pl.when(kk + 2 < NCH2)
            def _():
                _load_idx(kk + 2, lax.rem(kk + 2, 3))

            pltpu.make_async_copy(
                h_hbm.at[gidx.at[p]], rows.at[p], gsem.at[p]).wait()

            for g in range(C2 // 16):
                s16 = idxb[r, 0, pl.ds(g * 16, 16)]
                d16 = idxb[r, 1, pl.ds(g * 16, 16)]
                a = (plsc.load_gather(asrc_t, [s16])
                     + plsc.load_gather(adst_t, [d16])
                     + plsc.bitcast(idxb[r, 2, pl.ds(g * 16, 16)], _f32))
                a = jnp.maximum(a, 0.2 * a)
                wv = jnp.exp(a)
                wbuf[pl.ds(g * 16, 16)] = wv
                plsc.addupdate_scatter(den_t, [d16], wv)

            UN = 8
            def _scale(e8, _):
                base = e8 * UN
                ws = [plsc.load_gather(wbuf, [jnp.full((16,), base + u, _i32)])
                      for u in range(UN)]
                for u in range(UN):
                    for j in range(F2 // 16):
                        rows[p, base + u, pl.ds(j * 16, 16)] = (
                            rows[p, base + u, pl.ds(j * 16, 16)] * ws[u])
                return 0
            lax.fori_loop(0, C2 // UN, _scale, 0)

            pltpu.async_copy(rows.at[p], num_sh.at[idxb.at[r, 1]], ssem.at[p],
                             add=True)
            return 0
        lax.fori_loop(0, NCH2, _chunk, 0)

        for b in (0, 1):
            pltpu.make_async_copy(
                rows.at[b], num_sh.at[idxb.at[0, 1]], ssem.at[b]).wait()

        plsc.subcore_barrier()
        pltpu.sync_copy(num_sh.at[pl.ds(sid * RPT, RPT)],
                        num_out.at[cid, pl.ds(sid * RPT, RPT)])
        pltpu.sync_copy(den_t, den_out.at[w])

    return k(hcat, asrc, adst, eidx)


def _tc_matmul_scores(x, W, UV, F, interpret=False):
    BN = 2048
    grid = (N_PAD // BN,)

    def body(x_ref, w_ref, uv_ref, h2_ref, sc_ref):
        xb = x_ref[...]
        hb = jnp.dot(xb, w_ref[...], preferred_element_type=_f32)
        F2 = hb.shape[1] // 2
        h2_ref[0] = hb[:, :F2]
        h2_ref[1] = hb[:, F2:]
        sc_ref[...] = lax.dot_general(
            uv_ref[...], xb, (((1,), (1,)), ((), ())),
            preferred_element_type=_f32)

    return pl.pallas_call(
        body,
        grid=grid,
        in_specs=[
            pl.BlockSpec((BN, x.shape[1]), lambda i: (i, 0)),
            pl.BlockSpec(W.shape, lambda i: (0, 0)),
            pl.BlockSpec(UV.shape, lambda i: (0, 0)),
        ],
        out_specs=[
            pl.BlockSpec((2, BN, F // 2), lambda i: (0, i, 0)),
            pl.BlockSpec((8, BN), lambda i: (0, i)),
        ],
        out_shape=[
            jax.ShapeDtypeStruct((2, N_PAD, F // 2), _f32),
            jax.ShapeDtypeStruct((8, N_PAD), _f32),
        ],
        interpret=interpret,
    )(x, W, UV)


def _tc_loop_scores(lsum, lcnt, Vs, interpret=False):
    BN = 2048
    grid = (N_PAD // BN,)

    def body(ls_ref, lc_ref, vs_ref, out_ref):
        ls = ls_ref[0] + ls_ref[1]
        lc = jnp.sum(lc_ref[...], axis=0)
        la = ls / jnp.maximum(lc, 1.0)[:, None]
        out_ref[...] = lax.dot_general(
            vs_ref[...], la, (((1,), (1,)), ((), ())),
            preferred_element_type=_f32)

    return pl.pallas_call(
        body,
        grid=grid,
        in_specs=[
            pl.BlockSpec((2, BN, DE), lambda i: (0, i, 0)),
            pl.BlockSpec((NTILES, BN), lambda i: (0, i)),
            pl.BlockSpec(Vs.shape, lambda i: (0, 0)),
        ],
        out_specs=pl.BlockSpec((8, BN), lambda i: (0, i)),
        out_shape=jax.ShapeDtypeStruct((8, N_PAD), _f32),
        interpret=interpret,
    )(lsum, lcnt, Vs)


def _tc_edge_scores(edge_attr, V3, interpret=False):
    BE = 16384
    grid = (E_PAD // BE,)

    def body(ea_ref, v_ref, out_ref):
        out_ref[...] = lax.dot_general(
            v_ref[...], ea_ref[...], (((1,), (1,)), ((), ())),
            preferred_element_type=_f32)

    return pl.pallas_call(
        body,
        grid=grid,
        in_specs=[
            pl.BlockSpec((BE, DE), lambda i: (i, 0)),
            pl.BlockSpec(V3.shape, lambda i: (0, 0)),
        ],
        out_specs=pl.BlockSpec((8, BE), lambda i: (0, i)),
        out_shape=jax.ShapeDtypeStruct((8, E_PAD), _f32),
        interpret=interpret,
    )(edge_attr, V3)


def _tc_epilogue(num, den, h, scores, aloopT, b, lidx, F, interpret=False):
    BN = 2048
    grid = (N_PAD // BN,)

    def body(num_ref, den_ref, h_ref, sc_ref, al_ref, b_ref, out_ref):
        a = sc_ref[0] + sc_ref[1] + al_ref[lidx]
        a = jnp.maximum(a, 0.2 * a)
        wl = jnp.exp(a)
        hb = jnp.concatenate([h_ref[0], h_ref[1]], axis=1)
        nm = (jnp.concatenate([num_ref[0], num_ref[1]], axis=1)
              + wl[:, None] * hb)
        dn = 0.5 * jnp.sum(den_ref[...], axis=0) + wl
        out = nm / dn[:, None] + b_ref[...]
        out_ref[...] = jnp.maximum(out, 0.0)

    return pl.pallas_call(
        body,
        grid=grid,
        in_specs=[
            pl.BlockSpec((2, BN, F // 2), lambda i: (0, i, 0)),
            pl.BlockSpec((NTILES, BN), lambda i: (0, i)),
            pl.BlockSpec((2, BN, F // 2), lambda i: (0, i, 0)),
            pl.BlockSpec((8, BN), lambda i: (0, i)),
            pl.BlockSpec((8, BN), lambda i: (0, i)),
            pl.BlockSpec((1, F), lambda i: (0, 0)),
        ],
        out_specs=pl.BlockSpec((BN, F), lambda i: (i, 0)),
        out_shape=jax.ShapeDtypeStruct((N_PAD, F), _f32),
        interpret=interpret,
    )(num, den, h, scores, aloopT, b)


def _tc_pool(h, batch3, interpret=False):
    BN = 1280
    grid_n = N_PAD // BN

    def body(h_ref, b_ref, out_ref, sacc, cacc):
        i = pl.program_id(0)
        bb = b_ref[0, 0, :]
        oh = (bb[:, None] == lax.broadcasted_iota(_i32, (BN, G), 1)).astype(_f32)
        s = lax.dot_general(oh, h_ref[...], (((0,), (0,)), ((), ())),
                            preferred_element_type=_f32)
        ones = jnp.ones((BN, NCLS), _f32)
        c = lax.dot_general(oh, ones, (((0,), (0,)), ((), ())),
                            preferred_element_type=_f32)

        @pl.when(i == 0)
        def _():
            sacc[...] = s
            cacc[...] = c

        @pl.when(i > 0)
        def _():
            sacc[...] = sacc[...] + s
            cacc[...] = cacc[...] + c

        @pl.when(i == grid_n - 1)
        def _():
            pooled = sacc[...] / jnp.maximum(cacc[...], 1.0)
            out_ref[...] = 1.0 / (1.0 + jnp.exp(-pooled))

    return pl.pallas_call(
        body,
        grid=(grid_n,),
        in_specs=[
            pl.BlockSpec((BN, NCLS), lambda i: (i, 0)),
            pl.BlockSpec((1, 1, BN), lambda i: (i, 0, 0)),
        ],
        out_specs=pl.BlockSpec((G, NCLS), lambda i: (0, 0)),
        out_shape=jax.ShapeDtypeStruct((G, NCLS), _f32),
        scratch_shapes=[
            pltpu.VMEM((G, NCLS), _f32),
            pltpu.VMEM((G, NCLS), _f32),
        ],
        interpret=interpret,
    )(h, batch3)


def _run(x, edge_index, edge_attr, batch, params, interpret=False):
    pad_i = jnp.full((E_PAD - E,), N_PAD - 1, _i32)
    src = jnp.concatenate([edge_index[0], pad_i])
    dst_p = jnp.concatenate([edge_index[1], pad_i])
    didx = dst_p.reshape(NTILES, NCH, C)
    ea_p = jnp.zeros((E_PAD, DE), _f32).at[:E].set(edge_attr)

    uvs = []
    V3 = jnp.zeros((8, DE), _f32)
    for l, (Wl, asl, adl, Wel, ael, bl) in enumerate(params):
        UV = (jnp.zeros((8, Wl.shape[0]), _f32)
              .at[0].set(Wl @ asl).at[1].set(Wl @ adl))
        uvs.append(UV)
        V3 = V3.at[l].set(Wel @ ael)

    xp = jnp.zeros((N_PAD, F_IN), _f32).at[:N].set(x)
    batch_p = jnp.full((N_PAD,), G, _i32).at[:N].set(batch)
    batch3 = batch_p.reshape(N_PAD // 1280, 1, 1280)

    lsum, lcnt = _sc_edge_attr_mean(ea_p, didx, interpret=interpret)
    aloopT = _tc_loop_scores(lsum, lcnt, V3, interpret=interpret)
    aeT = _tc_edge_scores(ea_p, V3, interpret=interpret)

    h_in = xp
    for l, (Wl, asl, adl, Wel, ael, bl) in enumerate(params):
        F = Wl.shape[1]
        h2, scores = _tc_matmul_scores(h_in, Wl, uvs[l], F, interpret=interpret)
        hcat = h2.reshape(2 * N_PAD, F // 2)
        ae_bits = lax.bitcast_convert_type(aeT[l], _i32)
        eidx = (jnp.stack([src, dst_p, ae_bits])
                .reshape(3, 16, NCH2, C2).transpose(1, 2, 0, 3))
        num, den = _sc_edge_aggregate(hcat, scores[0], scores[1], eidx, F,
                                      interpret=interpret)
        h_in = _tc_epilogue(num, den, h2, scores, aloopT, bl.reshape(1, F),
                            l, F, interpret=interpret)

    return _tc_pool(h_in, batch3, interpret=interpret)


def kernel(x, edge_index, edge_attr, batch, W1, as1, ad1, We1, ae1, b1,
           W2, as2, ad2, We2, ae2, b2, W3, as3, ad3, We3, ae3, b3):
    params = [
        (W1, as1, ad1, We1, ae1, b1),
        (W2, as2, ad2, We2, ae2, b2),
        (W3, as3, ad3, We3, ae3, b3),
    ]
    return _run(x, edge_index, edge_attr, batch, params)


# speedup vs baseline: 1.3400x; 1.0430x over previous
import functools

import jax
import jax.numpy as jnp
from jax import lax
from jax.experimental import pallas as pl
from jax.experimental.pallas import tpu as pltpu
from jax.experimental.pallas import tpu_sc as plsc

N = 10000
E = 320000
F_IN = 128
HID = 128
NCLS = 32
DE = 16
G = 64

N_PAD = 10240
E_PAD = 327680
C = 64
NTILES = 32
EPT = E_PAD // NTILES
NCH = EPT // C
RPT = N_PAD // 16
C2 = 128
EPT2 = E_PAD // 16
NCH2 = EPT2 // C2

_f32 = jnp.float32
_i32 = jnp.int32


def _sc_mesh():
    return plsc.VectorSubcoreMesh(core_axis_name="c", subcore_axis_name="s",
                                  num_cores=2, num_subcores=16)


def _sc_edge_attr_mean(edge_attr, didx, interpret=False):
    kfn = functools.partial(
        pl.kernel,
        out_type=(
            jax.ShapeDtypeStruct((2, N_PAD, DE), _f32),
            jax.ShapeDtypeStruct((NTILES, N_PAD), _f32),
        ),
        mesh=_sc_mesh(),
        compiler_params=pltpu.CompilerParams(needs_layout_passes=False, use_tc_tiling_on_sc=False),
        scratch_types=[
            pltpu.VMEM((NCH, C), _i32),
            pltpu.VMEM((2, C, DE), _f32),
            pltpu.VMEM((N_PAD,), _f32),
            pltpu.VMEM((16,), _f32),
            pltpu.MemorySpace.VMEM_SHARED((N_PAD, DE), _f32),
            pltpu.SemaphoreType.DMA((2,)),
            pltpu.SemaphoreType.DMA((2,)),
        ],
        interpret=interpret,
    )

    @kfn
    def k(ea_hbm, didx_hbm, lsum_out, lcnt_out,
          didx_t, ebuf, cnt_t, ones_v, lsum_sh, lsem, ssemA):
        cid = lax.axis_index("c")
        sid = lax.axis_index("s")
        w = cid * 16 + sid

        def _z_ebuf(i, _):
            ebuf[0, i, :] = jnp.zeros((DE,), _f32)
            return 0
        lax.fori_loop(0, C, _z_ebuf, 0)

        def _z_cnt(i, _):
            cnt_t[pl.ds(i * 16, 16)] = jnp.zeros((16,), _f32)
            return 0
        lax.fori_loop(0, N_PAD // 16, _z_cnt, 0)
        ones_v[...] = jnp.ones((16,), _f32)

        def _z_sh(i, _):
            pltpu.sync_copy(ebuf.at[0],
                            lsum_sh.at[pl.ds(sid * RPT + i * C, C)])
            return 0
        lax.fori_loop(0, RPT // C, _z_sh, 0)
        plsc.subcore_barrier()

        pltpu.sync_copy(didx_hbm.at[w], didx_t)

        def _load_ea(kk, b):
            pltpu.async_copy(ea_hbm.at[pl.ds(w * EPT + kk * C, C)],
                             ebuf.at[b], lsem.at[b])

        _load_ea(0, 0)

        def _chunk(kk, _):
            p = lax.rem(kk, 2)
            q = 1 - p

            @pl.when(kk + 1 < NCH)
            def _():
                @pl.when(kk >= 1)
                def _():
                    pltpu.make_async_copy(
                        ebuf.at[q], lsum_sh.at[didx_t.at[kk]],
                        ssemA.at[q]).wait()
                _load_ea(kk + 1, q)

            pltpu.make_async_copy(
                ea_hbm.at[pl.ds(w * EPT + kk * C, C)], ebuf.at[p],
                lsem.at[p]).wait()
            for g in range(C // 16):
                d16 = didx_t[kk, pl.ds(g * 16, 16)]
                plsc.addupdate_scatter(cnt_t, [d16], ones_v[...])
            pltpu.async_copy(ebuf.at[p], lsum_sh.at[didx_t.at[kk]],
                             ssemA.at[p], add=True)
            return 0
        lax.fori_loop(0, NCH, _chunk, 0)

        for b in (0, 1):
            pltpu.make_async_copy(
                ebuf.at[b], lsum_sh.at[didx_t.at[0]], ssemA.at[b]).wait()

        plsc.subcore_barrier()
        pltpu.sync_copy(lsum_sh.at[pl.ds(sid * RPT, RPT)],
                        lsum_out.at[cid, pl.ds(sid * RPT, RPT)])
        pltpu.sync_copy(cnt_t, lcnt_out.at[w])

    return k(edge_attr, didx)


def _sc_edge_aggregate(hcat, asrc, adst, eidx, F, interpret=False):
    F2 = F // 2
    kfn = functools.partial(
        pl.kernel,
        out_type=(
            jax.ShapeDtypeStruct((2, N_PAD, F2), _f32),
            jax.ShapeDtypeStruct((NTILES, N_PAD), _f32),
        ),
        mesh=_sc_mesh(),
        compiler_params=pltpu.CompilerParams(needs_layout_passes=False, use_tc_tiling_on_sc=False),
        scratch_types=[
            pltpu.VMEM((N_PAD,), _f32),
            pltpu.VMEM((N_PAD,), _f32),
            pltpu.VMEM((N_PAD,), _f32),
            pltpu.VMEM((3, 3, C2), _i32),
            pltpu.VMEM((2, C2), _i32),
            pltpu.VMEM((2, C2, F2), _f32),
            pltpu.VMEM((C2,), _f32),
            pltpu.MemorySpace.VMEM_SHARED((N_PAD, F2), _f32),
            pltpu.SemaphoreType.DMA((3,)),
            pltpu.SemaphoreType.DMA((2,)),
            pltpu.SemaphoreType.DMA((2,)),
        ],
        interpret=interpret,
    )

    @kfn
    def k(h_hbm, asrc_hbm, adst_hbm, eidx_hbm,
          num_out, den_out,
          asrc_t, adst_t, den_t, idxb, gidx, rows, wbuf,
          num_sh, isem, gsem, ssem):
        cid = lax.axis_index("c")
        sid = lax.axis_index("s")
        w = cid * 16 + sid
        bias = cid * N_PAD

        def _z_rows(i, _):
            for j in range(F2 // 16):
                rows[0, i, pl.ds(j * 16, 16)] = jnp.zeros((16,), _f32)
            return 0
        lax.fori_loop(0, C2, _z_rows, 0)

        def _z_den(i, _):
            den_t[pl.ds(i * 16, 16)] = jnp.zeros((16,), _f32)
            return 0
        lax.fori_loop(0, N_PAD // 16, _z_den, 0)

        def _z_sh(i, _):
            pltpu.sync_copy(rows.at[0],
                            num_sh.at[pl.ds(sid * RPT + i * C2, C2)])
            return 0
        lax.fori_loop(0, RPT // C2, _z_sh, 0)
        plsc.subcore_barrier()

        pltpu.sync_copy(asrc_hbm, asrc_t)
        pltpu.sync_copy(adst_hbm, adst_t)

        def _load_idx(kk, r):
            pltpu.async_copy(eidx_hbm.at[sid, kk], idxb.at[r], isem.at[r])

        def _bias_idx(r, b):
            for j in range(C2 // 16):
                gidx[b, pl.ds(j * 16, 16)] = (
                    idxb[r, 0, pl.ds(j * 16, 16)] + bias)

        _load_idx(0, 0)
        _load_idx(1, 1)
        pltpu.make_async_copy(eidx_hbm.at[sid, 0], idxb.at[0], isem.at[0]).wait()
        _bias_idx(0, 0)
        pltpu.async_copy(h_hbm.at[gidx.at[0]], rows.at[0], gsem.at[0])

        def _chunk(kk, _):
            p = lax.rem(kk, 2)
            q = 1 - p
            r = lax.rem(kk, 3)

            @pl.when(kk + 1 < NCH2)
            def _():
                r1 = lax.rem(kk + 1, 3)
                pltpu.make_async_copy(
                    eidx_hbm.at[sid, kk], idxb.at[r1], isem.at[r1]).wait()

                @pl.when(kk >= 1)
                def _():
                    pltpu.make_async_copy(
                        rows.at[q], num_sh.at[idxb.at[r1, 1]],
                        ssem.at[q]).wait()
                _bias_idx(r1, q)
                pltpu.async_copy(h_hbm.at[gidx.at[q]], rows.at[q],
                                 gsem.at[q])

            @pl.when(kk + 2 < NCH2)
            def _():
                _load_idx(kk + 2, lax.rem(kk + 2, 3))

            pltpu.make_async_copy(
                h_hbm.at[gidx.at[p]], rows.at[p], gsem.at[p]).wait()

            for g in range(C2 // 16):
                s16 = idxb[r, 0, pl.ds(g * 16, 16)]
                d16 = idxb[r, 1, pl.ds(g * 16, 16)]
                a = (plsc.load_gather(asrc_t, [s16])
                     + plsc.load_gather(adst_t, [d16])
                     + plsc.bitcast(idxb[r, 2, pl.ds(g * 16, 16)], _f32))
                a = jnp.maximum(a, 0.2 * a)
                wv = jnp.exp(a)
                wbuf[pl.ds(g * 16, 16)] = wv
                plsc.addupdate_scatter(den_t, [d16], wv)

            UN = 8
            def _scale(e8, _):
                base = e8 * UN
                ws = [plsc.load_gather(wbuf, [jnp.full((16,), base + u, _i32)])
                      for u in range(UN)]
                for u in range(UN):
                    for j in range(F2 // 16):
                        rows[p, base + u, pl.ds(j * 16, 16)] = (
                            rows[p, base + u, pl.ds(j * 16, 16)] * ws[u])
                return 0
            lax.fori_loop(0, C2 // UN, _scale, 0)

            pltpu.async_copy(rows.at[p], num_sh.at[idxb.at[r, 1]], ssem.at[p],
                             add=True)
            return 0
        lax.fori_loop(0, NCH2, _chunk, 0)

        for b in (0, 1):
            pltpu.make_async_copy(
                rows.at[b], num_sh.at[idxb.at[0, 1]], ssem.at[b]).wait()

        plsc.subcore_barrier()
        pltpu.sync_copy(num_sh.at[pl.ds(sid * RPT, RPT)],
                        num_out.at[cid, pl.ds(sid * RPT, RPT)])
        pltpu.sync_copy(den_t, den_out.at[w])

    return k(hcat, asrc, adst, eidx)


def _tc_matmul_scores(x, W, UV, F, interpret=False):
    BN = 2048
    grid = (N_PAD // BN,)

    def body(x_ref, w_ref, uv_ref, h2_ref, sc_ref):
        xb = x_ref[...]
        hb = jnp.dot(xb, w_ref[...], preferred_element_type=_f32)
        F2 = hb.shape[1] // 2
        h2_ref[0] = hb[:, :F2]
        h2_ref[1] = hb[:, F2:]
        sc_ref[...] = lax.dot_general(
            uv_ref[...], xb, (((1,), (1,)), ((), ())),
            preferred_element_type=_f32)

    return pl.pallas_call(
        body,
        grid=grid,
        in_specs=[
            pl.BlockSpec((BN, x.shape[1]), lambda i: (i, 0)),
            pl.BlockSpec(W.shape, lambda i: (0, 0)),
            pl.BlockSpec(UV.shape, lambda i: (0, 0)),
        ],
        out_specs=[
            pl.BlockSpec((2, BN, F // 2), lambda i: (0, i, 0)),
            pl.BlockSpec((8, BN), lambda i: (0, i)),
        ],
        out_shape=[
            jax.ShapeDtypeStruct((2, N_PAD, F // 2), _f32),
            jax.ShapeDtypeStruct((8, N_PAD), _f32),
        ],
        interpret=interpret,
    )(x, W, UV)


def _tc_loop_scores(lsum, lcnt, Vs, interpret=False):
    BN = 2048
    grid = (N_PAD // BN,)

    def body(ls_ref, lc_ref, vs_ref, out_ref):
        ls = ls_ref[0] + ls_ref[1]
        lc = jnp.sum(lc_ref[...], axis=0)
        la = ls / jnp.maximum(lc, 1.0)[:, None]
        out_ref[...] = lax.dot_general(
            vs_ref[...], la, (((1,), (1,)), ((), ())),
            preferred_element_type=_f32)

    return pl.pallas_call(
        body,
        grid=grid,
        in_specs=[
            pl.BlockSpec((2, BN, DE), lambda i: (0, i, 0)),
            pl.BlockSpec((NTILES, BN), lambda i: (0, i)),
            pl.BlockSpec(Vs.shape, lambda i: (0, 0)),
        ],
        out_specs=pl.BlockSpec((8, BN), lambda i: (0, i)),
        out_shape=jax.ShapeDtypeStruct((8, N_PAD), _f32),
        interpret=interpret,
    )(lsum, lcnt, Vs)


def _tc_edge_scores(edge_attr, V3, interpret=False):
    BE = 16384
    grid = (E_PAD // BE,)

    def body(ea_ref, v_ref, out_ref):
        out_ref[...] = lax.dot_general(
            v_ref[...], ea_ref[...], (((1,), (1,)), ((), ())),
            preferred_element_type=_f32)

    return pl.pallas_call(
        body,
        grid=grid,
        in_specs=[
            pl.BlockSpec((BE, DE), lambda i: (i, 0)),
            pl.BlockSpec(V3.shape, lambda i: (0, 0)),
        ],
        out_specs=pl.BlockSpec((8, BE), lambda i: (0, i)),
        out_shape=jax.ShapeDtypeStruct((8, E_PAD), _f32),
        interpret=interpret,
    )(edge_attr, V3)


def _tc_epilogue(num, den, h, scores, aloopT, b, lidx, F, interpret=False):
    BN = 2048
    grid = (N_PAD // BN,)

    def body(num_ref, den_ref, h_ref, sc_ref, al_ref, b_ref, out_ref):
        a = sc_ref[0] + sc_ref[1] + al_ref[lidx]
        a = jnp.maximum(a, 0.2 * a)
        wl = jnp.exp(a)
        hb = jnp.concatenate([h_ref[0], h_ref[1]], axis=1)
        nm = (jnp.concatenate([num_ref[0], num_ref[1]], axis=1)
              + wl[:, None] * hb)
        dn = 0.5 * jnp.sum(den_ref[...], axis=0) + wl
        out = nm / dn[:, None] + b_ref[...]
        out_ref[...] = jnp.maximum(out, 0.0)

    return pl.pallas_call(
        body,
        grid=grid,
        in_specs=[
            pl.BlockSpec((2, BN, F // 2), lambda i: (0, i, 0)),
            pl.BlockSpec((NTILES, BN), lambda i: (0, i)),
            pl.BlockSpec((2, BN, F // 2), lambda i: (0, i, 0)),
            pl.BlockSpec((8, BN), lambda i: (0, i)),
            pl.BlockSpec((8, BN), lambda i: (0, i)),
            pl.BlockSpec((1, F), lambda i: (0, 0)),
        ],
        out_specs=pl.BlockSpec((BN, F), lambda i: (i, 0)),
        out_shape=jax.ShapeDtypeStruct((N_PAD, F), _f32),
        interpret=interpret,
    )(num, den, h, scores, aloopT, b)


def _tc_pool(h, batch3, interpret=False):
    BN = 1280
    grid_n = N_PAD // BN

    def body(h_ref, b_ref, out_ref, sacc, cacc):
        i = pl.program_id(0)
        bb = b_ref[0, 0, :]
        oh = (bb[:, None] == lax.broadcasted_iota(_i32, (BN, G), 1)).astype(_f32)
        s = lax.dot_general(oh, h_ref[...], (((0,), (0,)), ((), ())),
                            preferred_element_type=_f32)
        ones = jnp.ones((BN, NCLS), _f32)
        c = lax.dot_general(oh, ones, (((0,), (0,)), ((), ())),
                            preferred_element_type=_f32)

        @pl.when(i == 0)
        def _():
            sacc[...] = s
            cacc[...] = c

        @pl.when(i > 0)
        def _():
            sacc[...] = sacc[...] + s
            cacc[...] = cacc[...] + c

        @pl.when(i == grid_n - 1)
        def _():
            pooled = sacc[...] / jnp.maximum(cacc[...], 1.0)
            out_ref[...] = 1.0 / (1.0 + jnp.exp(-pooled))

    return pl.pallas_call(
        body,
        grid=(grid_n,),
        in_specs=[
            pl.BlockSpec((BN, NCLS), lambda i: (i, 0)),
            pl.BlockSpec((1, 1, BN), lambda i: (i, 0, 0)),
        ],
        out_specs=pl.BlockSpec((G, NCLS), lambda i: (0, 0)),
        out_shape=jax.ShapeDtypeStruct((G, NCLS), _f32),
        scratch_shapes=[
            pltpu.VMEM((G, NCLS), _f32),
            pltpu.VMEM((G, NCLS), _f32),
        ],
        interpret=interpret,
    )(h, batch3)


def _run(x, edge_index, edge_attr, batch, params, interpret=False):
    pad_i = jnp.full((E_PAD - E,), N_PAD - 1, _i32)
    src = jnp.concatenate([edge_index[0], pad_i])
    dst_p = jnp.concatenate([edge_index[1], pad_i])
    didx = dst_p.reshape(NTILES, NCH, C)
    ea_p = jnp.zeros((E_PAD, DE), _f32).at[:E].set(edge_attr)

    uvs = []
    V3 = jnp.zeros((8, DE), _f32)
    for l, (Wl, asl, adl, Wel, ael, bl) in enumerate(params):
        UV = (jnp.zeros((8, Wl.shape[0]), _f32)
              .at[0].set(Wl @ asl).at[1].set(Wl @ adl))
        uvs.append(UV)
        V3 = V3.at[l].set(Wel @ ael)

    xp = jnp.zeros((N_PAD, F_IN), _f32).at[:N].set(x)
    batch_p = jnp.full((N_PAD,), G, _i32).at[:N].set(batch)
    batch3 = batch_p.reshape(N_PAD // 1280, 1, 1280)

    lsum, lcnt = _sc_edge_attr_mean(ea_p, didx, interpret=interpret)
    aloopT = _tc_loop_scores(lsum, lcnt, V3, interpret=interpret)
    aeT = _tc_edge_scores(ea_p, V3, interpret=interpret)

    h_in = xp
    for l, (Wl, asl, adl, Wel, ael, bl) in enumerate(params):
        F = Wl.shape[1]
        h2, scores = _tc_matmul_scores(h_in, Wl, uvs[l], F, interpret=interpret)
        hcat = h2.reshape(2 * N_PAD, F // 2)
        ae_bits = lax.bitcast_convert_type(aeT[l], _i32)
        eidx = (jnp.stack([src, dst_p, ae_bits])
                .reshape(3, 16, NCH2, C2).transpose(1, 2, 0, 3))
        num, den = _sc_edge_aggregate(hcat, scores[0], scores[1], eidx, F,
                                      interpret=interpret)
        h_in = _tc_epilogue(num, den, h2, scores, aloopT, bl.reshape(1, F),
                            l, F, interpret=interpret)

    return _tc_pool(h_in, batch3, interpret=interpret)


def kernel(x, edge_index, edge_attr, batch, W1, as1, ad1, We1, ae1, b1,
           W2, as2, ad2, We2, ae2, b2, W3, as3, ad3, We3, ae3, b3):
    params = [
        (W1, as1, ad1, We1, ae1, b1),
        (W2, as2, ad2, We2, ae2, b2),
        (W3, as3, ad3, We3, ae3, b3),
    ]
    return _run(x, edge_index, edge_attr, batch, params)
